# baseline (speedup 1.0000x reference)
import functools

import jax
import jax.numpy as jnp
from jax import lax
from jax.experimental import pallas as pl
from jax.experimental.pallas import tpu as pltpu
from jax.experimental.pallas import tpu_sc as plsc

VOCAB = 1000
D = 2048
B = 4
S = 4096
NTOK = B * S

NC = 2
NS = 16
NW = NC * NS
TOK_PER_W = NTOK // NW
CHUNK = 16
NCHUNK = TOK_PER_W // CHUNK


def _proj_body(emb_ref, w_ref, b_ref, out_ref):
    acc = lax.dot_general(
        emb_ref[...], w_ref[...],
        dimension_numbers=(((1,), (1,)), ((), ())),
        preferred_element_type=jnp.float32,
    )
    out_ref[...] = acc + b_ref[...]


_NBLK = 16


def _project_table(emb, W, bias):
    blk = D // _NBLK
    return pl.pallas_call(
        _proj_body,
        grid=(_NBLK,),
        in_specs=[
            pl.BlockSpec((VOCAB, D), lambda i: (0, 0)),
            pl.BlockSpec((blk, D), lambda i: (i, 0)),
            pl.BlockSpec((1, blk), lambda i: (0, i)),
        ],
        out_specs=pl.BlockSpec((VOCAB, blk), lambda i: (0, i)),
        out_shape=jax.ShapeDtypeStruct((VOCAB, D), jnp.float32),
    )(emb, W, bias.reshape(1, D))


CH = 8
NB = 7
NCH = TOK_PER_W // CH
PRIME = NB - 1


def _gather_body(x_hbm, table_hbm, out_hbm, idx_v, rows_v,
                 gsem, *ssem):
    cid = lax.axis_index("c")
    sid = lax.axis_index("s")
    wid = sid * NC + cid
    pltpu.sync_copy(x_hbm.at[wid], idx_v)

    def gstart(c, buf):
        pltpu.async_copy(table_hbm.at[idx_v.at[pl.ds(c * CH, CH)]],
                         rows_v.at[buf], gsem)

    def gwait():
        pltpu.make_async_copy(table_hbm.at[idx_v.at[pl.ds(0, CH)]],
                              rows_v.at[0], gsem).wait()

    def sstart(c, buf):
        pltpu.async_copy(rows_v.at[buf],
                         out_hbm.at[wid, pl.ds(c * CH, CH)], ssem[buf])

    def swait(buf):
        pltpu.make_async_copy(rows_v.at[0],
                              out_hbm.at[wid, pl.ds(0, CH)], ssem[buf]).wait()

    for c in range(PRIME):
        gstart(c, c)
    gwait(); sstart(0, 0); gstart(PRIME, PRIME)

    def steady(g, carry):
        for j in range(NB):
            c = g * NB + j + 1
            b = (j + 1) % NB
            gwait(); sstart(c, b); swait((b + NB - 1) % NB)
            gstart(c + PRIME, (b + PRIME) % NB)
        return carry

    lax.fori_loop(0, (NCH - PRIME - 2) // NB, steady, 0)

    c0 = ((NCH - PRIME - 2) // NB) * NB + 1
    gwait(); sstart(c0, c0 % NB); swait((c0 - 1) % NB); gstart(c0 + PRIME, (c0 + PRIME) % NB)
    for c in range(c0 + 1, NCH):
        gwait(); sstart(c, c % NB); swait((c - 1) % NB)
    swait((NCH - 1) % NB)


_gather = functools.partial(
    pl.kernel,
    out_type=jax.ShapeDtypeStruct((NW, TOK_PER_W, D), jnp.float32),
    mesh=plsc.VectorSubcoreMesh(
        core_axis_name="c", subcore_axis_name="s",
        num_cores=NC, num_subcores=NS),
    scratch_types=[
        pltpu.VMEM((TOK_PER_W,), jnp.int32),
        pltpu.VMEM((NB, CH, D), jnp.float32),
        pltpu.SemaphoreType.DMA,
    ] + [pltpu.SemaphoreType.DMA] * NB,
)(_gather_body)


def kernel(x, emb, W, b):
    proj = _project_table(emb, W, b)
    idx = x.reshape(NW, TOK_PER_W)
    out = _gather(idx, proj)
    return out.reshape(B, S, D)


# speedup vs baseline: 1.0770x; 1.0770x over previous
import functools

import jax
import jax.numpy as jnp
from jax import lax
from jax.experimental import pallas as pl
from jax.experimental.pallas import tpu as pltpu
from jax.experimental.pallas import tpu_sc as plsc

VOCAB = 1000
D = 2048
B = 4
S = 4096
NTOK = B * S

NC = 2
NS = 16
NW = NC * NS
TOK_PER_W = NTOK // NW
CHUNK = 16
NCHUNK = TOK_PER_W // CHUNK


def _proj_body(emb_ref, w_ref, b_ref, out_ref):
    acc = lax.dot_general(
        emb_ref[...], w_ref[...],
        dimension_numbers=(((1,), (1,)), ((), ())),
        preferred_element_type=jnp.float32,
    )
    out_ref[...] = acc + b_ref[...]


_NBLK = 4


def _project_table(emb, W, bias):
    blk = D // _NBLK
    return pl.pallas_call(
        _proj_body,
        grid=(_NBLK,),
        in_specs=[
            pl.BlockSpec((VOCAB, D), lambda i: (0, 0)),
            pl.BlockSpec((blk, D), lambda i: (i, 0)),
            pl.BlockSpec((1, blk), lambda i: (0, i)),
        ],
        out_specs=pl.BlockSpec((VOCAB, blk), lambda i: (0, i)),
        out_shape=jax.ShapeDtypeStruct((VOCAB, D), jnp.float32),
    )(emb, W, bias.reshape(1, D))


CH = 8
NB = 7
NCH = TOK_PER_W // CH
PRIME = NB - 1


def _gather_body(x_hbm, table_hbm, out_hbm, idx_v, rows_v,
                 gsem, *ssem):
    cid = lax.axis_index("c")
    sid = lax.axis_index("s")
    wid = sid * NC + cid
    pltpu.sync_copy(x_hbm.at[wid], idx_v)

    def gstart(c, buf):
        pltpu.async_copy(table_hbm.at[idx_v.at[pl.ds(c * CH, CH)]],
                         rows_v.at[buf], gsem)

    def gwait():
        pltpu.make_async_copy(table_hbm.at[idx_v.at[pl.ds(0, CH)]],
                              rows_v.at[0], gsem).wait()

    def sstart(c, buf):
        pltpu.async_copy(rows_v.at[buf],
                         out_hbm.at[wid, pl.ds(c * CH, CH)], ssem[buf])

    def swait(buf):
        pltpu.make_async_copy(rows_v.at[0],
                              out_hbm.at[wid, pl.ds(0, CH)], ssem[buf]).wait()

    for c in range(PRIME):
        gstart(c, c)
    gwait(); sstart(0, 0); gstart(PRIME, PRIME)

    def steady(g, carry):
        for j in range(NB):
            c = g * NB + j + 1
            b = (j + 1) % NB
            gwait(); sstart(c, b); swait((b + NB - 1) % NB)
            gstart(c + PRIME, (b + PRIME) % NB)
        return carry

    lax.fori_loop(0, (NCH - PRIME - 2) // NB, steady, 0)

    c0 = ((NCH - PRIME - 2) // NB) * NB + 1
    gwait(); sstart(c0, c0 % NB); swait((c0 - 1) % NB); gstart(c0 + PRIME, (c0 + PRIME) % NB)
    for c in range(c0 + 1, NCH):
        gwait(); sstart(c, c % NB); swait((c - 1) % NB)
    swait((NCH - 1) % NB)


_gather = functools.partial(
    pl.kernel,
    out_type=jax.ShapeDtypeStruct((NW, TOK_PER_W, D), jnp.float32),
    mesh=plsc.VectorSubcoreMesh(
        core_axis_name="c", subcore_axis_name="s",
        num_cores=NC, num_subcores=NS),
    scratch_types=[
        pltpu.VMEM((TOK_PER_W,), jnp.int32),
        pltpu.VMEM((NB, CH, D), jnp.float32),
        pltpu.SemaphoreType.DMA,
    ] + [pltpu.SemaphoreType.DMA] * NB,
)(_gather_body)


def kernel(x, emb, W, b):
    proj = _project_table(emb, W, b)
    idx = x.reshape(NW, TOK_PER_W)
    out = _gather(idx, proj)
    return out.reshape(B, S, D)
